# baseline (speedup 1.0000x reference)
import jax
import jax.numpy as jnp
from jax.experimental import pallas as pl


def _qgps_body(s_ref, e0_ref, e1_ref, o_ref):
    sf = s_ref[...].astype(jnp.float32)
    e0 = e0_ref[...]
    e1 = e1_ref[...]
    t0 = jnp.maximum(jnp.log(jnp.abs(e0)), -1e4)
    t1 = jnp.maximum(jnp.log(jnp.abs(e1)), -1e4)
    n0 = (e0 < 0).astype(jnp.float32)
    n1 = (e1 < 0).astype(jnp.float32)
    rhs = jnp.concatenate([t1 - t0, n1 - n0], axis=1)
    base = jnp.concatenate([jnp.sum(t0, axis=0, keepdims=True),
                            jnp.sum(n0, axis=0, keepdims=True)], axis=1)
    acc = base + jax.lax.dot(sf, rhs,
                             preferred_element_type=jnp.float32)
    n = e0_ref.shape[1]
    logp = acc[:, :n]
    negs = acc[:, n:]
    sign = 1.0 - 2.0 * (negs - 2.0 * jnp.floor(negs * 0.5))
    psi = sign * jnp.exp(logp)
    o_ref[...] = jnp.sum(psi, axis=1, keepdims=True)


def kernel(inputs, eps):
    if inputs.ndim == 1:
        inputs = jnp.expand_dims(inputs, axis=0)
    B, L = inputs.shape
    N = eps.shape[1]
    s8 = inputs.astype(jnp.int8)
    e0 = eps[0].T
    e1 = eps[1].T
    out = pl.pallas_call(
        _qgps_body,
        in_specs=[
            pl.BlockSpec((B, L), lambda: (0, 0)),
            pl.BlockSpec((L, N), lambda: (0, 0)),
            pl.BlockSpec((L, N), lambda: (0, 0)),
        ],
        out_specs=pl.BlockSpec((B, 1), lambda: (0, 0)),
        out_shape=jax.ShapeDtypeStruct((B, 1), jnp.float32),
    )(s8, e0, e1)
    return out.reshape(B)


# speedup vs baseline: 1.8634x; 1.8634x over previous
import jax
import jax.numpy as jnp
from jax.experimental import pallas as pl

_DN = (((1,), (1,)), ((), ()))


def _qgps_body(s_ref, e_ref, o_ref):
    sf = s_ref[...].astype(jnp.float32)
    e0 = e_ref[0]
    e1 = e_ref[1]
    t0 = jnp.maximum(jnp.log(jnp.abs(e0)), -1e4)
    t1 = jnp.maximum(jnp.log(jnp.abs(e1)), -1e4)
    n0 = (e0 < 0).astype(jnp.float32)
    n1 = (e1 < 0).astype(jnp.float32)
    rhs = jnp.concatenate([t1 - t0, n1 - n0], axis=0)
    ref0 = jnp.concatenate([t0, n0], axis=0)
    ones = jnp.ones((1, ref0.shape[1]), jnp.float32)
    base = jax.lax.dot_general(ones, ref0, _DN,
                               preferred_element_type=jnp.float32)
    acc = base + jax.lax.dot_general(sf, rhs, _DN,
                                     preferred_element_type=jnp.float32)
    n = e_ref.shape[1]
    logp = acc[:, :n]
    negs = acc[:, n:]
    sign = 1.0 - 2.0 * (negs - 2.0 * jnp.floor(negs * 0.5))
    psi = sign * jnp.exp(logp)
    o_ref[...] = jnp.sum(psi, axis=1, keepdims=True)


def kernel(inputs, eps):
    if inputs.ndim == 1:
        inputs = jnp.expand_dims(inputs, axis=0)
    B, L = inputs.shape
    N = eps.shape[1]
    out = pl.pallas_call(
        _qgps_body,
        in_specs=[
            pl.BlockSpec((B, L), lambda: (0, 0)),
            pl.BlockSpec((2, N, L), lambda: (0, 0, 0)),
        ],
        out_specs=pl.BlockSpec((B, 1), lambda: (0, 0)),
        out_shape=jax.ShapeDtypeStruct((B, 1), jnp.float32),
    )(inputs, eps)
    return out.reshape(B)
